# Initial kernel scaffold; baseline (speedup 1.0000x reference)
import jax
import jax.numpy as jnp
from jax.experimental import pallas as pl


def kernel(x, edge_index, edge_attr, params):
    raise NotImplementedError("write your pallas kernel here")


# speedup vs baseline: 1.6825x; 1.6825x over previous
import functools

import jax
import jax.numpy as jnp
from jax import lax
from jax.experimental import pallas as pl
from jax.experimental.pallas import tpu as pltpu
from jax.experimental.pallas import tpu_sc as plsc

N_NODES = 10000
N_EDGES = 320000
NPAD = 10240
NPG = NPAD // 16
W = 128
CHUNK = 128
BS = 2048
F32 = jnp.float32


def _sc_dims():
    try:
        info = plsc.get_sparse_core_info()
        return int(info.num_cores), int(info.num_subcores)
    except Exception:
        return 2, 16


def _make_gather2(nc, ns, cpw):
    nw = nc * ns
    epad = nw * cpw * CHUNK
    mesh = plsc.VectorSubcoreMesh(core_axis_name="c", subcore_axis_name="s")
    grp = 4

    @functools.partial(
        pl.kernel,
        out_type=(jax.ShapeDtypeStruct((epad, W), F32),
                  jax.ShapeDtypeStruct((epad, W), F32)),
        mesh=mesh,
        scratch_types=[pltpu.VMEM((cpw, CHUNK), jnp.int32),
                       pltpu.VMEM((cpw, CHUNK), jnp.int32),
                       pltpu.VMEM((grp, CHUNK, W), F32),
                       pltpu.SemaphoreType.DMA],
    )
    def gather2(table_hbm, idxa_hbm, idxb_hbm, outa_hbm, outb_hbm,
                idxa_v, idxb_v, rows_v, sem):
        wid = lax.axis_index("s") * nc + lax.axis_index("c")
        r0 = wid * cpw
        pltpu.sync_copy(idxa_hbm.at[pl.ds(r0, cpw)], idxa_v)
        pltpu.sync_copy(idxb_hbm.at[pl.ds(r0, cpw)], idxb_v)
        for idx_v, out_hbm in ((idxa_v, outa_hbm), (idxb_v, outb_hbm)):
            @pl.loop(0, cpw, step=grp)
            def _(g, idx_v=idx_v, out_hbm=out_hbm):
                for j in range(grp):
                    pltpu.async_copy(table_hbm.at[idx_v.at[g + j]],
                                     rows_v.at[j], sem)
                for j in range(grp):
                    pltpu.make_async_copy(table_hbm.at[idx_v.at[g + j]],
                                          rows_v.at[j], sem).wait()
                for j in range(grp):
                    pltpu.sync_copy(
                        rows_v.at[j],
                        out_hbm.at[pl.ds((r0 + g + j) * CHUNK, CHUNK)])

    return gather2


def _make_scatter(nc, ns, cpw):
    rpt = NPG // ns
    mesh = plsc.VectorSubcoreMesh(core_axis_name="c", subcore_axis_name="s")
    grp = 4

    @functools.partial(
        pl.kernel,
        out_type=jax.ShapeDtypeStruct((nc, NPG, W), F32),
        mesh=mesh,
        scratch_types=[pltpu.VMEM((cpw, CHUNK), jnp.int32),
                       pltpu.VMEM((grp, CHUNK, W), F32),
                       pltpu.VMEM_SHARED((NPG, W), F32),
                       pltpu.SemaphoreType.DMA],
    )
    def scatter(msg_hbm, idx_hbm, zero_hbm, out_hbm, idx_v, msg_v, acc_sh, sem):
        cid = lax.axis_index("c")
        sid = lax.axis_index("s")
        wid = sid * nc + cid
        r0 = wid * cpw
        t0 = sid * rpt
        pltpu.sync_copy(zero_hbm.at[pl.ds(t0, rpt)], acc_sh.at[pl.ds(t0, rpt)])
        pltpu.sync_copy(idx_hbm.at[pl.ds(r0, cpw)], idx_v)
        plsc.subcore_barrier()

        @pl.loop(0, cpw, step=grp)
        def _(g):
            for j in range(grp):
                pltpu.async_copy(msg_hbm.at[pl.ds((r0 + g + j) * CHUNK, CHUNK)],
                                 msg_v.at[j], sem)
            for j in range(grp):
                pltpu.make_async_copy(
                    msg_hbm.at[pl.ds((r0 + g + j) * CHUNK, CHUNK)],
                    msg_v.at[j], sem).wait()
            for j in range(grp):
                pltpu.sync_copy(msg_v.at[j], acc_sh.at[idx_v.at[g + j]],
                                add=True)

        plsc.subcore_barrier()
        pltpu.sync_copy(acc_sh.at[pl.ds(t0, rpt)],
                        out_hbm.at[cid, pl.ds(t0, rpt)])

    return scatter


def _full_spec(shape):
    nd = len(shape)
    return pl.BlockSpec(shape, lambda i, _nd=nd: (0,) * _nd)


def _make_msg(cin, cout, epad, kdim):
    grid = epad // BS

    def body(ea_ref, hs_ref, dm_ref, w1, b1, w2, b2, w3, b3, st_ref, out_ref):
        dot = functools.partial(jnp.dot, preferred_element_type=F32)
        h = jnp.maximum(dot(ea_ref[...], w1[...]) + b1[...], 0.0)
        h = jnp.maximum(dot(h, w2[...]) + b2[...], 0.0)
        w = jnp.maximum(dot(h, w3[...]) + b3[...], 0.0)
        m = st_ref[0:1, :]
        inv = st_ref[1:2, :]
        hn = (hs_ref[...] - m) * inv
        if cin == 1:
            msg = hn[:, 0:1] * w
        elif cout == 8:
            msg = hn[:, 0:1] * w[:, 0:8]
            for i in range(1, 8):
                msg = msg + hn[:, i:i + 1] * w[:, i * 8:(i + 1) * 8]
        else:
            s = hn[:, 0:1] * w[:, 0:1]
            for i in range(1, 8):
                s = s + hn[:, i:i + 1] * w[:, i:i + 1]
            lane = lax.broadcasted_iota(jnp.int32, (BS, 8), 1)
            msg = jnp.where(lane == 0, s, 0.0)
        grp16 = lax.broadcasted_iota(jnp.int32, (BS, W), 1) // 8
        out_ref[...] = jnp.where(grp16 == dm_ref[...],
                                 jnp.concatenate([msg] * 16, axis=1), 0.0)

    return pl.pallas_call(
        body,
        grid=(grid,),
        in_specs=[
            pl.BlockSpec((BS, 2), lambda i: (i, 0)),
            pl.BlockSpec((BS, W), lambda i: (i, 0)),
            pl.BlockSpec((BS, 1), lambda i: (i, 0)),
            _full_spec((2, 4)), _full_spec((1, 4)),
            _full_spec((4, 16)), _full_spec((1, 16)),
            _full_spec((16, kdim)), _full_spec((1, kdim)),
            _full_spec((2, W)),
        ],
        out_specs=pl.BlockSpec((BS, W), lambda i: (i, 0)),
        out_shape=jax.ShapeDtypeStruct((epad, W), F32),
    )


def _make_stats():
    def body(x_ref, st_ref):
        x = x_ref[...]
        m = jnp.sum(x, axis=0, keepdims=True) / N_NODES
        v = jnp.sum(x * x, axis=0, keepdims=True) / N_NODES - m * m
        st_ref[...] = jnp.concatenate([m, lax.rsqrt(v + 1e-5)], axis=0)

    return pl.pallas_call(
        body, out_shape=jax.ShapeDtypeStruct((2, W), F32))


def _make_node(resid):
    def body(x_ref, agg_ref, root_ref, bias_ref, out_ref, st_ref):
        x = x_ref[...]
        m = jnp.sum(x, axis=0, keepdims=True) / N_NODES
        v = jnp.sum(x * x, axis=0, keepdims=True) / N_NODES - m * m
        hn = (x - m) * lax.rsqrt(v + 1e-5)
        agg = (agg_ref[0] + agg_ref[1])[:, 0:8]
        y = agg + jnp.dot(hn, root_ref[...], preferred_element_type=F32)
        y = jnp.maximum(y + bias_ref[...], 0.0)
        if resid == "full":
            y = y + x[:, 0:8]
        elif resid == "bcast":
            y = y + x[:, 0:1]
        row = lax.broadcasted_iota(jnp.int32, (NPAD, 8), 0)
        y = jnp.where(row < N_NODES, y, 0.0)
        yw = jnp.concatenate([y, jnp.zeros((NPAD, W - 8), F32)], axis=1)
        out_ref[...] = yw
        m2 = jnp.sum(yw, axis=0, keepdims=True) / N_NODES
        v2 = jnp.sum(yw * yw, axis=0, keepdims=True) / N_NODES - m2 * m2
        st_ref[...] = jnp.concatenate([m2, lax.rsqrt(v2 + 1e-5)], axis=0)

    return pl.pallas_call(
        body,
        out_shape=(jax.ShapeDtypeStruct((NPAD, W), F32),
                   jax.ShapeDtypeStruct((2, W), F32)),
    )


def _make_edge(din, resid, epad):
    grid = epad // BS

    def body(ea_ref, xs_ref, xd_ref, w1s, w1d, w1e, b1, g_ref, bb_ref,
             w2, b2, out_ref):
        dot = functools.partial(jnp.dot, preferred_element_type=F32)
        ea = ea_ref[...]
        if din == 18:
            h = dot(xs_ref[...], w1s[...]) + dot(xd_ref[...], w1d[...])
        else:
            h = dot(xs_ref[:, 0:1], w1s[...]) + dot(xd_ref[:, 0:1], w1d[...])
        h = jnp.maximum(h + dot(ea, w1e[...]) + b1[...], 0.0)
        mu = jnp.mean(h, axis=1, keepdims=True)
        va = jnp.mean(h * h, axis=1, keepdims=True) - mu * mu
        hln = (h - mu) * lax.rsqrt(va + 1e-5) * g_ref[...] + bb_ref[...]
        o = jnp.maximum(dot(hln, w2[...]) + b2[...], 0.0)
        if resid:
            o = o + ea
        out_ref[...] = o

    nsrc = W if din == 18 else 1
    return pl.pallas_call(
        body,
        grid=(grid,),
        in_specs=[
            pl.BlockSpec((BS, 2), lambda i: (i, 0)),
            pl.BlockSpec((BS, W), lambda i: (i, 0)),
            pl.BlockSpec((BS, W), lambda i: (i, 0)),
            _full_spec((nsrc, 8)), _full_spec((nsrc, 8)), _full_spec((2, 8)),
            _full_spec((1, 8)), _full_spec((1, 8)), _full_spec((1, 8)),
            _full_spec((8, 2)), _full_spec((1, 2)),
        ],
        out_specs=pl.BlockSpec((BS, 2), lambda i: (i, 0)),
        out_shape=jax.ShapeDtypeStruct((epad, 2), F32),
    )


def _nn_weights(p, kdim):
    nn = p["nn"]
    return (nn["l1"]["W"], nn["l1"]["b"].reshape(1, 4),
            nn["l2"]["W"], nn["l2"]["b"].reshape(1, 16),
            nn["l3"]["W"], nn["l3"]["b"].reshape(1, kdim))


def _edge_weights(p, din):
    w1 = p["l1"]["W"]
    if din == 18:
        w1s = jnp.zeros((W, 8), F32).at[0:8].set(w1[0:8])
        w1d = jnp.zeros((W, 8), F32).at[0:8].set(w1[8:16])
        w1e = w1[16:18]
    else:
        w1s, w1d, w1e = w1[0:1], w1[1:2], w1[2:4]
    w2 = p["l2"]["W"]
    b2 = p["l2"]["b"]
    if w2.shape[1] == 1:
        w2 = jnp.concatenate([w2, jnp.zeros((8, 1), F32)], axis=1)
        b2 = jnp.concatenate([b2, jnp.zeros((1,), F32)])
    return (w1s, w1d, w1e, p["l1"]["b"].reshape(1, 8),
            p["ln_g"].reshape(1, 8), p["ln_b"].reshape(1, 8),
            w2, b2.reshape(1, 2))


def kernel(x, edge_index, edge_attr, params):
    nc, ns = _sc_dims()
    nw = nc * ns
    cpw = -(-N_EDGES // (nw * CHUNK))
    cpw = -(-cpw // 4) * 4
    epad = nw * cpw * CHUNK

    src = edge_index[0].astype(jnp.int32)
    dst = edge_index[1].astype(jnp.int32)
    pad_idx = N_NODES + jnp.arange(epad - N_EDGES, dtype=jnp.int32) % (
        NPAD - N_NODES)
    src_m = jnp.concatenate([src, pad_idx]).reshape(-1, CHUNK)
    dst_pad = jnp.concatenate([dst, pad_idx])
    dst_m = dst_pad.reshape(-1, CHUNK)
    dstg_m = (dst_pad // 16).reshape(-1, CHUNK)
    dstmod = (dst_pad % 16).reshape(-1, 1)
    ea = jnp.zeros((epad, 2), F32).at[:N_EDGES].set(edge_attr)
    x0 = jnp.zeros((NPAD, W), F32).at[:N_NODES, 0:1].set(x)
    zero_node = jnp.zeros((NPG, W), F32)

    gather2 = _make_gather2(nc, ns, cpw)
    scatter = _make_scatter(nc, ns, cpw)
    msg_in = _make_msg(1, 8, epad, 8)
    msg_mid = _make_msg(8, 8, epad, 64)
    msg_out = _make_msg(8, 1, epad, 8)
    node_bcast = _make_node("bcast")
    node_full = _make_node("full")
    node_none = _make_node("none")
    edge18 = _make_edge(18, True, epad)
    edge4 = _make_edge(4, False, epad)
    stats = _make_stats()

    def conv(x_cur, xs_cur, st_cur, ea_cur, p, msg_k, node_k, root, bias):
        kdim = p["nn"]["l3"]["W"].shape[1]
        msg = msg_k(ea_cur, xs_cur, dstmod, *_nn_weights(p, kdim), st_cur)
        agg = scatter(msg, dstg_m, zero_node)
        agg = agg.reshape(nc, NPAD, 8)
        return node_k(x_cur, agg, root, bias)

    st0 = stats(x0)
    xs0, _ = gather2(x0, src_m, dst_m)
    p = params["conv_in"]
    root = jnp.zeros((W, 8), F32).at[0:1].set(p["root"])
    x1, st1 = conv(x0, xs0, st0, ea, p, msg_in, node_bcast,
                   root, p["bias"].reshape(1, 8))
    xs, xd = gather2(x1, src_m, dst_m)
    ea1 = edge18(ea, xs, xd, *_edge_weights(params["edge_in"], 18))

    x_cur, st_cur, ea_cur, xs_cur = x1, st1, ea1, xs
    for i in range(4):
        p = params["convs"][i]
        root = jnp.zeros((W, 8), F32).at[0:8].set(p["root"])
        x_new, st_new = conv(x_cur, xs_cur, st_cur, ea_cur, p, msg_mid,
                             node_full, root, p["bias"].reshape(1, 8))
        xs, xd = gather2(x_new, src_m, dst_m)
        ea_new = edge18(ea_cur, xs, xd,
                        *_edge_weights(params["edge_convs"][i], 18))
        x_cur, st_cur, ea_cur, xs_cur = x_new, st_new, ea_new, xs

    p = params["conv_out"]
    root = jnp.zeros((W, 8), F32).at[0:8, 0:1].set(p["root"])
    bias = jnp.zeros((1, 8), F32).at[0, 0:1].set(p["bias"])
    x_f, _ = conv(x_cur, xs_cur, st_cur, ea_cur, p, msg_out, node_none,
                  root, bias)
    xs, xd = gather2(x_f, src_m, dst_m)
    ea_f = edge4(ea_cur, xs, xd, *_edge_weights(params["edge_out"], 4))

    return (x_f[:N_NODES, 0:1], ea_f[:N_EDGES, 0:1])


# speedup vs baseline: 1.6937x; 1.0066x over previous
import functools

import jax
import jax.numpy as jnp
from jax import lax
from jax.experimental import pallas as pl
from jax.experimental.pallas import tpu as pltpu
from jax.experimental.pallas import tpu_sc as plsc

N_NODES = 10000
N_EDGES = 320000
NPAD = 10240
NPG = NPAD // 16
W = 128
CHUNK = 128
BS = 2048
F32 = jnp.float32


def _sc_dims():
    try:
        info = plsc.get_sparse_core_info()
        return int(info.num_cores), int(info.num_subcores)
    except Exception:
        return 2, 16


def _make_gather2(nc, ns, cpw):
    nw = nc * ns
    epad = nw * cpw * CHUNK
    mesh = plsc.VectorSubcoreMesh(core_axis_name="c", subcore_axis_name="s")
    grp = 4

    @functools.partial(
        pl.kernel,
        out_type=(jax.ShapeDtypeStruct((epad, W), F32),
                  jax.ShapeDtypeStruct((epad, W), F32)),
        mesh=mesh,
        scratch_types=[pltpu.VMEM((cpw, CHUNK), jnp.int32),
                       pltpu.VMEM((cpw, CHUNK), jnp.int32),
                       pltpu.VMEM((grp, CHUNK, W), F32),
                       pltpu.SemaphoreType.DMA],
    )
    def gather2(table_hbm, idxa_hbm, idxb_hbm, outa_hbm, outb_hbm,
                idxa_v, idxb_v, rows_v, sem):
        wid = lax.axis_index("s") * nc + lax.axis_index("c")
        r0 = wid * cpw
        pltpu.sync_copy(idxa_hbm.at[pl.ds(r0, cpw)], idxa_v)
        pltpu.sync_copy(idxb_hbm.at[pl.ds(r0, cpw)], idxb_v)
        for idx_v, out_hbm in ((idxa_v, outa_hbm), (idxb_v, outb_hbm)):
            @pl.loop(0, cpw, step=grp)
            def _(g, idx_v=idx_v, out_hbm=out_hbm):
                for j in range(grp):
                    pltpu.async_copy(table_hbm.at[idx_v.at[g + j]],
                                     rows_v.at[j], sem)
                for j in range(grp):
                    pltpu.make_async_copy(table_hbm.at[idx_v.at[g + j]],
                                          rows_v.at[j], sem).wait()
                for j in range(grp):
                    pltpu.sync_copy(
                        rows_v.at[j],
                        out_hbm.at[pl.ds((r0 + g + j) * CHUNK, CHUNK)])

    return gather2


def _make_gather1(nc, ns, cpw):
    nw = nc * ns
    epad = nw * cpw * CHUNK
    mesh = plsc.VectorSubcoreMesh(core_axis_name="c", subcore_axis_name="s")
    grp = 4

    @functools.partial(
        pl.kernel,
        out_type=jax.ShapeDtypeStruct((epad, W), F32),
        mesh=mesh,
        scratch_types=[pltpu.VMEM((cpw, CHUNK), jnp.int32),
                       pltpu.VMEM((grp, CHUNK, W), F32),
                       pltpu.SemaphoreType.DMA],
    )
    def gather1(table_hbm, idx_hbm, out_hbm, idx_v, rows_v, sem):
        wid = lax.axis_index("s") * nc + lax.axis_index("c")
        r0 = wid * cpw
        pltpu.sync_copy(idx_hbm.at[pl.ds(r0, cpw)], idx_v)

        @pl.loop(0, cpw, step=grp)
        def _(g):
            for j in range(grp):
                pltpu.async_copy(table_hbm.at[idx_v.at[g + j]],
                                 rows_v.at[j], sem)
            for j in range(grp):
                pltpu.make_async_copy(table_hbm.at[idx_v.at[g + j]],
                                      rows_v.at[j], sem).wait()
            for j in range(grp):
                pltpu.sync_copy(
                    rows_v.at[j],
                    out_hbm.at[pl.ds((r0 + g + j) * CHUNK, CHUNK)])

    return gather1


def _make_scatter(nc, ns, cpw):
    rpt = NPG // ns
    mesh = plsc.VectorSubcoreMesh(core_axis_name="c", subcore_axis_name="s")
    grp = 4

    @functools.partial(
        pl.kernel,
        out_type=jax.ShapeDtypeStruct((nc, NPG, W), F32),
        mesh=mesh,
        scratch_types=[pltpu.VMEM((cpw, CHUNK), jnp.int32),
                       pltpu.VMEM((grp, CHUNK, W), F32),
                       pltpu.VMEM_SHARED((NPG, W), F32),
                       pltpu.SemaphoreType.DMA],
    )
    def scatter(msg_hbm, idx_hbm, zero_hbm, out_hbm, idx_v, msg_v, acc_sh, sem):
        cid = lax.axis_index("c")
        sid = lax.axis_index("s")
        wid = sid * nc + cid
        r0 = wid * cpw
        t0 = sid * rpt
        pltpu.sync_copy(zero_hbm.at[pl.ds(t0, rpt)], acc_sh.at[pl.ds(t0, rpt)])
        pltpu.sync_copy(idx_hbm.at[pl.ds(r0, cpw)], idx_v)
        plsc.subcore_barrier()

        @pl.loop(0, cpw, step=grp)
        def _(g):
            for j in range(grp):
                pltpu.async_copy(msg_hbm.at[pl.ds((r0 + g + j) * CHUNK, CHUNK)],
                                 msg_v.at[j], sem)
            for j in range(grp):
                pltpu.make_async_copy(
                    msg_hbm.at[pl.ds((r0 + g + j) * CHUNK, CHUNK)],
                    msg_v.at[j], sem).wait()
            for j in range(grp):
                pltpu.sync_copy(msg_v.at[j], acc_sh.at[idx_v.at[g + j]],
                                add=True)

        plsc.subcore_barrier()
        pltpu.sync_copy(acc_sh.at[pl.ds(t0, rpt)],
                        out_hbm.at[cid, pl.ds(t0, rpt)])

    return scatter


def _full_spec(shape):
    nd = len(shape)
    return pl.BlockSpec(shape, lambda i, _nd=nd: (0,) * _nd)


def _make_msg(cin, cout, epad, kdim):
    grid = epad // BS

    def body(ea_ref, hs_ref, dm_ref, w1, b1, w2, b2, w3, b3, st_ref, out_ref):
        dot = functools.partial(jnp.dot, preferred_element_type=F32)
        h = jnp.maximum(dot(ea_ref[...], w1[...]) + b1[...], 0.0)
        h = jnp.maximum(dot(h, w2[...]) + b2[...], 0.0)
        w = jnp.maximum(dot(h, w3[...]) + b3[...], 0.0)
        m = st_ref[0:1, :]
        inv = st_ref[1:2, :]
        hn = (hs_ref[...] - m) * inv
        if cin == 1:
            msg = hn[:, 0:1] * w
        elif cout == 8:
            msg = hn[:, 0:1] * w[:, 0:8]
            for i in range(1, 8):
                msg = msg + hn[:, i:i + 1] * w[:, i * 8:(i + 1) * 8]
        else:
            s = hn[:, 0:1] * w[:, 0:1]
            for i in range(1, 8):
                s = s + hn[:, i:i + 1] * w[:, i:i + 1]
            lane = lax.broadcasted_iota(jnp.int32, (BS, 8), 1)
            msg = jnp.where(lane == 0, s, 0.0)
        grp16 = lax.broadcasted_iota(jnp.int32, (BS, W), 1) // 8
        out_ref[...] = jnp.where(grp16 == dm_ref[...],
                                 jnp.concatenate([msg] * 16, axis=1), 0.0)

    return pl.pallas_call(
        body,
        grid=(grid,),
        in_specs=[
            pl.BlockSpec((BS, 2), lambda i: (i, 0)),
            pl.BlockSpec((BS, W), lambda i: (i, 0)),
            pl.BlockSpec((BS, 1), lambda i: (i, 0)),
            _full_spec((2, 4)), _full_spec((1, 4)),
            _full_spec((4, 16)), _full_spec((1, 16)),
            _full_spec((16, kdim)), _full_spec((1, kdim)),
            _full_spec((2, W)),
        ],
        out_specs=pl.BlockSpec((BS, W), lambda i: (i, 0)),
        out_shape=jax.ShapeDtypeStruct((epad, W), F32),
    )


def _make_stats():
    def body(x_ref, st_ref):
        x = x_ref[...]
        m = jnp.sum(x, axis=0, keepdims=True) / N_NODES
        v = jnp.sum(x * x, axis=0, keepdims=True) / N_NODES - m * m
        st_ref[...] = jnp.concatenate([m, lax.rsqrt(v + 1e-5)], axis=0)

    return pl.pallas_call(
        body, out_shape=jax.ShapeDtypeStruct((2, W), F32))


def _make_node(resid):
    def body(x_ref, agg_ref, root_ref, bias_ref, out_ref, st_ref):
        x = x_ref[...]
        m = jnp.sum(x, axis=0, keepdims=True) / N_NODES
        v = jnp.sum(x * x, axis=0, keepdims=True) / N_NODES - m * m
        hn = (x - m) * lax.rsqrt(v + 1e-5)
        agg = (agg_ref[0] + agg_ref[1])[:, 0:8]
        y = agg + jnp.dot(hn, root_ref[...], preferred_element_type=F32)
        y = jnp.maximum(y + bias_ref[...], 0.0)
        if resid == "full":
            y = y + x[:, 0:8]
        elif resid == "bcast":
            y = y + x[:, 0:1]
        row = lax.broadcasted_iota(jnp.int32, (NPAD, 8), 0)
        y = jnp.where(row < N_NODES, y, 0.0)
        yw = jnp.concatenate([y, jnp.zeros((NPAD, W - 8), F32)], axis=1)
        out_ref[...] = yw
        m2 = jnp.sum(yw, axis=0, keepdims=True) / N_NODES
        v2 = jnp.sum(yw * yw, axis=0, keepdims=True) / N_NODES - m2 * m2
        st_ref[...] = jnp.concatenate([m2, lax.rsqrt(v2 + 1e-5)], axis=0)

    return pl.pallas_call(
        body,
        out_shape=(jax.ShapeDtypeStruct((NPAD, W), F32),
                   jax.ShapeDtypeStruct((2, W), F32)),
    )


def _make_edge(din, resid, epad):
    grid = epad // BS

    def body(ea_ref, xs_ref, xd_ref, w1s, w1d, w1e, b1, g_ref, bb_ref,
             w2, b2, out_ref):
        dot = functools.partial(jnp.dot, preferred_element_type=F32)
        ea = ea_ref[...]
        if din == 18:
            h = dot(xs_ref[...], w1s[...]) + dot(xd_ref[...], w1d[...])
        else:
            h = dot(xs_ref[:, 0:1], w1s[...]) + dot(xd_ref[:, 0:1], w1d[...])
        h = jnp.maximum(h + dot(ea, w1e[...]) + b1[...], 0.0)
        mu = jnp.mean(h, axis=1, keepdims=True)
        va = jnp.mean(h * h, axis=1, keepdims=True) - mu * mu
        hln = (h - mu) * lax.rsqrt(va + 1e-5) * g_ref[...] + bb_ref[...]
        o = jnp.maximum(dot(hln, w2[...]) + b2[...], 0.0)
        if resid:
            o = o + ea
        out_ref[...] = o

    nsrc = W if din == 18 else 1
    return pl.pallas_call(
        body,
        grid=(grid,),
        in_specs=[
            pl.BlockSpec((BS, 2), lambda i: (i, 0)),
            pl.BlockSpec((BS, W), lambda i: (i, 0)),
            pl.BlockSpec((BS, W), lambda i: (i, 0)),
            _full_spec((nsrc, 8)), _full_spec((nsrc, 8)), _full_spec((2, 8)),
            _full_spec((1, 8)), _full_spec((1, 8)), _full_spec((1, 8)),
            _full_spec((8, 2)), _full_spec((1, 2)),
        ],
        out_specs=pl.BlockSpec((BS, 2), lambda i: (i, 0)),
        out_shape=jax.ShapeDtypeStruct((epad, 2), F32),
    )


def _nn_weights(p, kdim):
    nn = p["nn"]
    return (nn["l1"]["W"], nn["l1"]["b"].reshape(1, 4),
            nn["l2"]["W"], nn["l2"]["b"].reshape(1, 16),
            nn["l3"]["W"], nn["l3"]["b"].reshape(1, kdim))


def _edge_weights(p, din):
    w1 = p["l1"]["W"]
    if din == 18:
        w1s = jnp.zeros((W, 8), F32).at[0:8].set(w1[0:8])
        w1d = jnp.zeros((W, 8), F32).at[0:8].set(w1[8:16])
        w1e = w1[16:18]
    else:
        w1s, w1d, w1e = w1[0:1], w1[1:2], w1[2:4]
    w2 = p["l2"]["W"]
    b2 = p["l2"]["b"]
    if w2.shape[1] == 1:
        w2 = jnp.concatenate([w2, jnp.zeros((8, 1), F32)], axis=1)
        b2 = jnp.concatenate([b2, jnp.zeros((1,), F32)])
    return (w1s, w1d, w1e, p["l1"]["b"].reshape(1, 8),
            p["ln_g"].reshape(1, 8), p["ln_b"].reshape(1, 8),
            w2, b2.reshape(1, 2))


def kernel(x, edge_index, edge_attr, params):
    nc, ns = _sc_dims()
    nw = nc * ns
    cpw = -(-N_EDGES // (nw * CHUNK))
    cpw = -(-cpw // 4) * 4
    epad = nw * cpw * CHUNK

    src = edge_index[0].astype(jnp.int32)
    dst = edge_index[1].astype(jnp.int32)
    pad_idx = N_NODES + jnp.arange(epad - N_EDGES, dtype=jnp.int32) % (
        NPAD - N_NODES)
    src_m = jnp.concatenate([src, pad_idx]).reshape(-1, CHUNK)
    dst_pad = jnp.concatenate([dst, pad_idx])
    dst_m = dst_pad.reshape(-1, CHUNK)
    dstg_m = (dst_pad // 16).reshape(-1, CHUNK)
    dstmod = (dst_pad % 16).reshape(-1, 1)
    ea = jnp.zeros((epad, 2), F32).at[:N_EDGES].set(edge_attr)
    x0 = jnp.zeros((NPAD, W), F32).at[:N_NODES, 0:1].set(x)
    zero_node = jnp.zeros((NPG, W), F32)

    gather2 = _make_gather2(nc, ns, cpw)
    gather1 = _make_gather1(nc, ns, cpw)
    scatter = _make_scatter(nc, ns, cpw)
    msg_in = _make_msg(1, 8, epad, 8)
    msg_mid = _make_msg(8, 8, epad, 64)
    msg_out = _make_msg(8, 1, epad, 8)
    node_bcast = _make_node("bcast")
    node_full = _make_node("full")
    node_none = _make_node("none")
    edge18 = _make_edge(18, True, epad)
    edge4 = _make_edge(4, False, epad)
    stats = _make_stats()

    def conv(x_cur, xs_cur, st_cur, ea_cur, p, msg_k, node_k, root, bias):
        kdim = p["nn"]["l3"]["W"].shape[1]
        msg = msg_k(ea_cur, xs_cur, dstmod, *_nn_weights(p, kdim), st_cur)
        agg = scatter(msg, dstg_m, zero_node)
        agg = agg.reshape(nc, NPAD, 8)
        return node_k(x_cur, agg, root, bias)

    st0 = stats(x0)
    xs0 = gather1(x0, src_m)
    p = params["conv_in"]
    root = jnp.zeros((W, 8), F32).at[0:1].set(p["root"])
    x1, st1 = conv(x0, xs0, st0, ea, p, msg_in, node_bcast,
                   root, p["bias"].reshape(1, 8))
    xs, xd = gather2(x1, src_m, dst_m)
    ea1 = edge18(ea, xs, xd, *_edge_weights(params["edge_in"], 18))

    x_cur, st_cur, ea_cur, xs_cur = x1, st1, ea1, xs
    for i in range(4):
        p = params["convs"][i]
        root = jnp.zeros((W, 8), F32).at[0:8].set(p["root"])
        x_new, st_new = conv(x_cur, xs_cur, st_cur, ea_cur, p, msg_mid,
                             node_full, root, p["bias"].reshape(1, 8))
        xs, xd = gather2(x_new, src_m, dst_m)
        ea_new = edge18(ea_cur, xs, xd,
                        *_edge_weights(params["edge_convs"][i], 18))
        x_cur, st_cur, ea_cur, xs_cur = x_new, st_new, ea_new, xs

    p = params["conv_out"]
    root = jnp.zeros((W, 8), F32).at[0:8, 0:1].set(p["root"])
    bias = jnp.zeros((1, 8), F32).at[0, 0:1].set(p["bias"])
    x_f, _ = conv(x_cur, xs_cur, st_cur, ea_cur, p, msg_out, node_none,
                  root, bias)
    xs, xd = gather2(x_f, src_m, dst_m)
    ea_f = edge4(ea_cur, xs, xd, *_edge_weights(params["edge_out"], 4))

    return (x_f[:N_NODES, 0:1], ea_f[:N_EDGES, 0:1])
